# baseline (speedup 1.0000x reference)
import functools

import jax
import jax.numpy as jnp
from jax import lax
from jax.experimental import pallas as pl
from jax.experimental.pallas import tpu as pltpu
from jax.experimental.pallas import tpu_sc as plsc

_EMBEDDING_DIM = 64
_NUM_CODES = 1024
_BETA = 0.25


def _assign_kernel(x_ref, emb_ref,
                   enc_ref, idx_ref, embt_ref, ppl_ref,
                   counts_scr):
    i = pl.program_id(0)
    nsteps = pl.num_programs(0)
    blk_shape = x_ref.shape
    b = blk_shape[0] * blk_shape[1] * blk_shape[2]
    n_total = b * nsteps

    x = x_ref[...].reshape(b, _EMBEDDING_DIM)
    emb = emb_ref[...]

    @pl.when(i == 0)
    def _emit_embt():
        embt_ref[:, pl.ds(0, _EMBEDDING_DIM)] = emb.T
        embt_ref[:, pl.ds(_EMBEDDING_DIM, 128 - _EMBEDDING_DIM)] = (
            jnp.zeros((_NUM_CODES, 128 - _EMBEDDING_DIM), jnp.float32))

    esq = jnp.sum(emb * emb, axis=0, keepdims=True)
    xsq = jnp.sum(x * x, axis=1, keepdims=True)
    dist = (xsq + esq) - 2.0 * jnp.dot(
        x, emb, preferred_element_type=jnp.float32)

    idx = jnp.argmin(dist, axis=1)
    iota = jax.lax.broadcasted_iota(jnp.int32, (b, _NUM_CODES), 1)
    onehot = (iota == idx[:, None]).astype(jnp.float32)
    enc_ref[...] = onehot
    idx_ref[...] = idx.reshape(idx_ref.shape)

    prev_counts = jnp.where(i == 0, 0.0, counts_scr[...])
    counts_scr[...] = prev_counts + jnp.sum(onehot, axis=0, keepdims=True)

    @pl.when(i == nsteps - 1)
    def _finalize():
        avg = counts_scr[...] / n_total
        ent = jnp.sum(avg * jnp.log(avg + 1e-10)).reshape(1, 1)
        ppl_ref[...] = jnp.exp(-ent)


def _st_loss_kernel(x_ref, quant_ref, qst_ref, loss_ref, loss_scr):
    i = pl.program_id(0)
    nsteps = pl.num_programs(0)
    blk_shape = x_ref.shape
    b = blk_shape[0] * blk_shape[1] * blk_shape[2]
    n_total = b * nsteps

    x = x_ref[...].reshape(b, _EMBEDDING_DIM)
    quant = quant_ref[:, pl.ds(0, _EMBEDDING_DIM)]
    qst_ref[...] = (x + (quant - x)).reshape(blk_shape)

    diff = quant - x
    part = jnp.sum(diff * diff).reshape(1, 1)
    prev = jnp.where(i == 0, 0.0, loss_scr[...])
    loss_scr[...] = prev + part

    @pl.when(i == nsteps - 1)
    def _finalize():
        loss_ref[...] = _BETA * loss_scr[...] / (n_total * _EMBEDDING_DIM)


def _make_sc_gather(n_rows):
    info = plsc.get_sparse_core_info()
    nw = info.num_cores * info.num_subcores
    b_per_w = n_rows // nw
    mesh = plsc.VectorSubcoreMesh(core_axis_name="c", subcore_axis_name="s")

    @functools.partial(
        pl.kernel, mesh=mesh,
        out_type=jax.ShapeDtypeStruct((n_rows, 128), jnp.float32),
        scratch_types=[
            pltpu.VMEM((b_per_w,), jnp.int32),
            pltpu.VMEM((b_per_w, 128), jnp.float32),
            pltpu.SemaphoreType.DMA,
        ],
    )
    def gather(table_hbm, idx_hbm, out_hbm, idx_v, rows_v, sem):
        wid = lax.axis_index("s") * info.num_cores + lax.axis_index("c")
        base = wid * b_per_w
        pltpu.sync_copy(idx_hbm.at[pl.ds(base, b_per_w)], idx_v)
        pltpu.async_copy(table_hbm.at[idx_v], rows_v, sem).wait()
        pltpu.sync_copy(rows_v, out_hbm.at[pl.ds(base, b_per_w)])

    return gather


def kernel(x, embedding):
    batch, h, w, _ = x.shape
    n = batch * h * w
    imgs_per_blk = 2
    grid = (batch // imgs_per_blk,)
    block = imgs_per_blk * h * w

    enc, idx2d, emb_t, ppl = pl.pallas_call(
        _assign_kernel,
        grid=grid,
        in_specs=[
            pl.BlockSpec((imgs_per_blk, h, w, _EMBEDDING_DIM),
                         lambda i: (i, 0, 0, 0)),
            pl.BlockSpec((_EMBEDDING_DIM, _NUM_CODES), lambda i: (0, 0)),
        ],
        out_specs=[
            pl.BlockSpec((block, _NUM_CODES), lambda i: (i, 0)),
            pl.BlockSpec((block // 128, 128), lambda i: (i, 0)),
            pl.BlockSpec((_NUM_CODES, 128), lambda i: (0, 0)),
            pl.BlockSpec((1, 1), lambda i: (0, 0)),
        ],
        out_shape=[
            jax.ShapeDtypeStruct((n, _NUM_CODES), jnp.float32),
            jax.ShapeDtypeStruct((n // 128, 128), jnp.int32),
            jax.ShapeDtypeStruct((_NUM_CODES, 128), jnp.float32),
            jax.ShapeDtypeStruct((1, 1), jnp.float32),
        ],
        scratch_shapes=[
            pltpu.VMEM((1, _NUM_CODES), jnp.float32),
        ],
    )(x, embedding)

    idx_flat = idx2d.reshape(n)
    quant = _make_sc_gather(n)(emb_t, idx_flat)

    imgs_per_blk2 = 8
    grid2 = (batch // imgs_per_blk2,)
    block2 = imgs_per_blk2 * h * w
    qst, loss = pl.pallas_call(
        _st_loss_kernel,
        grid=grid2,
        in_specs=[
            pl.BlockSpec((imgs_per_blk2, h, w, _EMBEDDING_DIM),
                         lambda i: (i, 0, 0, 0)),
            pl.BlockSpec((block2, 128), lambda i: (i, 0)),
        ],
        out_specs=[
            pl.BlockSpec((imgs_per_blk2, h, w, _EMBEDDING_DIM),
                         lambda i: (i, 0, 0, 0)),
            pl.BlockSpec((1, 1), lambda i: (0, 0)),
        ],
        out_shape=[
            jax.ShapeDtypeStruct(x.shape, jnp.float32),
            jax.ShapeDtypeStruct((1, 1), jnp.float32),
        ],
        scratch_shapes=[
            pltpu.VMEM((1, 1), jnp.float32),
        ],
    )(x, quant)

    return (qst, loss.reshape(()), ppl.reshape(()), enc, idx_flat)


# speedup vs baseline: 2.3236x; 2.3236x over previous
import jax
import jax.numpy as jnp
from jax.experimental import pallas as pl
from jax.experimental.pallas import tpu as pltpu

_EMBEDDING_DIM = 64
_NUM_CODES = 1024
_BETA = 0.25


def _vq_kernel(x_ref, emb_ref,
               qst_ref, loss_ref, ppl_ref, enc_ref, idx_ref,
               counts_scr, loss_scr):
    i = pl.program_id(0)
    nsteps = pl.num_programs(0)
    blk_shape = x_ref.shape
    b = blk_shape[0] * blk_shape[1] * blk_shape[2]
    n_total = b * nsteps

    x = x_ref[...].reshape(b, _EMBEDDING_DIM)
    emb = emb_ref[...]

    esq = jnp.sum(emb * emb, axis=0, keepdims=True)
    xsq = jnp.sum(x * x, axis=1, keepdims=True)
    dist = (xsq + esq) - 2.0 * jnp.dot(
        x, emb, preferred_element_type=jnp.float32)

    idx = jnp.argmin(dist, axis=1)
    iota = jax.lax.broadcasted_iota(jnp.int32, (b, _NUM_CODES), 1)
    onehot = (iota == idx[:, None]).astype(jnp.float32)
    enc_ref[...] = onehot
    idx_ref[...] = idx.reshape(idx_ref.shape)

    quant = jax.lax.dot_general(
        onehot, emb, (((1,), (1,)), ((), ())),
        preferred_element_type=jnp.float32)
    qst_ref[...] = (x + (quant - x)).reshape(blk_shape)

    diff = quant - x
    part = jnp.sum(diff * diff).reshape(1, 1)
    prev_loss = jnp.where(i == 0, 0.0, loss_scr[...])
    loss_scr[...] = prev_loss + part

    prev_counts = jnp.where(i == 0, 0.0, counts_scr[...])
    counts_scr[...] = prev_counts + jnp.sum(onehot, axis=0, keepdims=True)

    @pl.when(i == nsteps - 1)
    def _finalize():
        loss_ref[...] = _BETA * loss_scr[...] / (n_total * _EMBEDDING_DIM)
        avg = counts_scr[...] / n_total
        ent = jnp.sum(avg * jnp.log(avg + 1e-10)).reshape(1, 1)
        ppl_ref[...] = jnp.exp(-ent)


def kernel(x, embedding):
    batch, h, w, _ = x.shape
    n = batch * h * w
    imgs_per_blk = 4
    block = imgs_per_blk * h * w
    grid = (batch // imgs_per_blk,)
    idx_rows_per_blk = block // 128

    qst, loss, ppl, enc, idx = pl.pallas_call(
        _vq_kernel,
        grid=grid,
        in_specs=[
            pl.BlockSpec((imgs_per_blk, h, w, _EMBEDDING_DIM),
                         lambda i: (i, 0, 0, 0)),
            pl.BlockSpec((_EMBEDDING_DIM, _NUM_CODES), lambda i: (0, 0)),
        ],
        out_specs=[
            pl.BlockSpec((imgs_per_blk, h, w, _EMBEDDING_DIM),
                         lambda i: (i, 0, 0, 0)),
            pl.BlockSpec((1, 1), lambda i: (0, 0)),
            pl.BlockSpec((1, 1), lambda i: (0, 0)),
            pl.BlockSpec((block, _NUM_CODES), lambda i: (i, 0)),
            pl.BlockSpec((idx_rows_per_blk, 128), lambda i: (i, 0)),
        ],
        out_shape=[
            jax.ShapeDtypeStruct(x.shape, jnp.float32),
            jax.ShapeDtypeStruct((1, 1), jnp.float32),
            jax.ShapeDtypeStruct((1, 1), jnp.float32),
            jax.ShapeDtypeStruct((n, _NUM_CODES), jnp.float32),
            jax.ShapeDtypeStruct((n // 128, 128), jnp.int32),
        ],
        scratch_shapes=[
            pltpu.VMEM((1, _NUM_CODES), jnp.float32),
            pltpu.VMEM((1, 1), jnp.float32),
        ],
    )(x, embedding)

    return (qst, loss.reshape(()), ppl.reshape(()), enc, idx.reshape(n))


# speedup vs baseline: 2.3694x; 1.0197x over previous
import jax
import jax.numpy as jnp
from jax.experimental import pallas as pl
from jax.experimental.pallas import tpu as pltpu

_EMBEDDING_DIM = 64
_NUM_CODES = 1024
_BETA = 0.25


def _vq_kernel(x_ref, emb_ref,
               qst_ref, loss_ref, ppl_ref, enc_ref, idx_ref,
               counts_scr, loss_scr):
    i = pl.program_id(0)
    nsteps = pl.num_programs(0)
    blk_shape = x_ref.shape
    b = blk_shape[0] * blk_shape[1] * blk_shape[2]
    n_total = b * nsteps

    x = x_ref[...].reshape(b, _EMBEDDING_DIM)
    emb = emb_ref[...]

    esq = jnp.sum(emb * emb, axis=0, keepdims=True)
    xsq = jnp.sum(x * x, axis=1, keepdims=True)
    dist = (xsq + esq) - 2.0 * jnp.dot(
        x, emb, preferred_element_type=jnp.float32)

    idx = jnp.argmin(dist, axis=1)
    iota = jax.lax.broadcasted_iota(jnp.int32, (1, _NUM_CODES), 1)
    onehot = (iota == idx[:, None]).astype(jnp.float32)
    enc_ref[...] = onehot
    idx_ref[...] = idx.reshape(idx_ref.shape)

    quant = jax.lax.dot_general(
        onehot, emb, (((1,), (1,)), ((), ())),
        preferred_element_type=jnp.float32)
    qst_ref[...] = (x + (quant - x)).reshape(blk_shape)

    diff = quant - x
    part = jnp.sum(diff * diff).reshape(1, 1)
    prev_loss = jnp.where(i == 0, 0.0, loss_scr[...])
    loss_scr[...] = prev_loss + part

    prev_counts = jnp.where(i == 0, 0.0, counts_scr[...])
    counts_scr[...] = prev_counts + jnp.sum(onehot, axis=0, keepdims=True)

    @pl.when(i == nsteps - 1)
    def _finalize():
        loss_ref[...] = _BETA * loss_scr[...] / (n_total * _EMBEDDING_DIM)
        avg = counts_scr[...] / n_total
        ent = jnp.sum(avg * jnp.log(avg + 1e-10)).reshape(1, 1)
        ppl_ref[...] = jnp.exp(-ent)


def kernel(x, embedding):
    batch, h, w, _ = x.shape
    n = batch * h * w
    imgs_per_blk = 2
    block = imgs_per_blk * h * w
    grid = (batch // imgs_per_blk,)
    idx_rows_per_blk = block // 128

    qst, loss, ppl, enc, idx = pl.pallas_call(
        _vq_kernel,
        grid=grid,
        in_specs=[
            pl.BlockSpec((imgs_per_blk, h, w, _EMBEDDING_DIM),
                         lambda i: (i, 0, 0, 0)),
            pl.BlockSpec((_EMBEDDING_DIM, _NUM_CODES), lambda i: (0, 0)),
        ],
        out_specs=[
            pl.BlockSpec((imgs_per_blk, h, w, _EMBEDDING_DIM),
                         lambda i: (i, 0, 0, 0)),
            pl.BlockSpec((1, 1), lambda i: (0, 0)),
            pl.BlockSpec((1, 1), lambda i: (0, 0)),
            pl.BlockSpec((block, _NUM_CODES), lambda i: (i, 0)),
            pl.BlockSpec((idx_rows_per_blk, 128), lambda i: (i, 0)),
        ],
        out_shape=[
            jax.ShapeDtypeStruct(x.shape, jnp.float32),
            jax.ShapeDtypeStruct((1, 1), jnp.float32),
            jax.ShapeDtypeStruct((1, 1), jnp.float32),
            jax.ShapeDtypeStruct((n, _NUM_CODES), jnp.float32),
            jax.ShapeDtypeStruct((n // 128, 128), jnp.int32),
        ],
        scratch_shapes=[
            pltpu.VMEM((1, _NUM_CODES), jnp.float32),
            pltpu.VMEM((1, 1), jnp.float32),
        ],
    )(x, embedding)

    return (qst, loss.reshape(()), ppl.reshape(()), enc, idx.reshape(n))
